# baseline (speedup 1.0000x reference)
import functools

import jax
import jax.numpy as jnp
from jax import lax
from jax.experimental import pallas as pl
from jax.experimental.pallas import tpu as pltpu
from jax.experimental.pallas import tpu_sc as plsc

_D = 32
_CHUNK = 256
_NW = 32
_TROWS = 1564
_OFF_BLACK = 768 * _D
_OFF_SMALL = (1536 * _D, 1540 * _D, 1544 * _D, 1552 * _D, 1560 * _D,
              1562 * _D)


def _sc_kernel(wpi_hbm, bpi_hbm, sm_hbm, table_hbm, out_hbm,
               table_v, wpi_v, bpi_v, sm_v, out_v, acc_v):
    B = out_hbm.shape[0] // _D
    per_w = B // _NW
    wid = lax.axis_index("s") * 2 + lax.axis_index("c")
    it16 = jnp.arange(16, dtype=jnp.int32)

    pltpu.sync_copy(table_hbm, table_v)

    def do_group(g, carry):
        lanes = g * 16 + it16

        for d in range(_D):
            acc_v[pl.ds(d * 16, 16)] = jnp.zeros((16,), jnp.float32)

        def make_body(idx_ref, off):
            def body(s, carry_):
                idxv = plsc.load_gather(idx_ref, [lanes * 64 + s])
                wordv = idxv * _D + (s * (12 * _D) + off)
                for d in range(_D):
                    val = plsc.load_gather(table_v, [wordv + d])
                    plsc.addupdate(acc_v.at[pl.ds(d * 16, 16)], val)
                return carry_
            return body

        lax.fori_loop(0, 64, make_body(wpi_v, 0), 0)
        lax.fori_loop(0, 64, make_body(bpi_v, _OFF_BLACK), 0)

        for s, off in enumerate(_OFF_SMALL):
            idxv = plsc.load_gather(sm_v, [lanes * 6 + s])
            wordv = idxv * _D + off
            for d in range(_D):
                val = plsc.load_gather(table_v, [wordv + d])
                plsc.addupdate(acc_v.at[pl.ds(d * 16, 16)], val)

        for d in range(_D):
            plsc.store_scatter(out_v, [lanes * _D + d],
                               acc_v[pl.ds(d * 16, 16)])
        return carry

    for chunk in range(per_w // _CHUNK):
        base = wid * per_w + chunk * _CHUNK
        pltpu.sync_copy(wpi_hbm.at[pl.ds(base * 64, _CHUNK * 64)], wpi_v)
        pltpu.sync_copy(bpi_hbm.at[pl.ds(base * 64, _CHUNK * 64)], bpi_v)
        pltpu.sync_copy(sm_hbm.at[pl.ds(base * 6, _CHUNK * 6)], sm_v)
        lax.fori_loop(0, _CHUNK // 16, do_group, 0)
        pltpu.sync_copy(out_v, out_hbm.at[pl.ds(base * _D, _CHUNK * _D)])


def kernel(white_piece_idx, black_piece_idx, white_castle_idx,
           black_castle_idx, white_ep_idx, black_ep_idx, white_fifty_idx,
           black_fifty_idx, W_white_piece, W_black_piece, W_white_castle,
           W_black_castle, W_white_ep, W_black_ep, W_white_fifty,
           W_black_fifty):
    B = white_piece_idx.shape[0]

    table = jnp.concatenate(
        [W_white_piece.reshape(768, _D), W_black_piece.reshape(768, _D),
         W_white_castle, W_black_castle, W_white_ep, W_black_ep,
         W_white_fifty, W_black_fifty], axis=0).reshape(-1)

    sm = jnp.stack(
        [white_castle_idx, black_castle_idx, white_ep_idx, black_ep_idx,
         white_fifty_idx, black_fifty_idx], axis=1).astype(jnp.int32)

    mesh = plsc.VectorSubcoreMesh(core_axis_name="c", subcore_axis_name="s")
    run = functools.partial(
        pl.kernel, mesh=mesh,
        compiler_params=pltpu.CompilerParams(needs_layout_passes=False),
        out_type=jax.ShapeDtypeStruct((B * _D,), jnp.float32),
        scratch_types=[
            pltpu.VMEM((_TROWS * _D,), jnp.float32),
            pltpu.VMEM((_CHUNK * 64,), jnp.int32),
            pltpu.VMEM((_CHUNK * 64,), jnp.int32),
            pltpu.VMEM((_CHUNK * 6,), jnp.int32),
            pltpu.VMEM((_CHUNK * _D,), jnp.float32),
            pltpu.VMEM((_D * 16,), jnp.float32),
        ],
    )(_sc_kernel)
    out = run(white_piece_idx.astype(jnp.int32).reshape(-1),
              black_piece_idx.astype(jnp.int32).reshape(-1),
              sm.reshape(-1), table)
    return out.reshape(B, _D)


# speedup vs baseline: 1.6825x; 1.6825x over previous
import functools

import jax
import jax.numpy as jnp
from jax import lax
from jax.experimental import pallas as pl
from jax.experimental.pallas import tpu as pltpu
from jax.experimental.pallas import tpu_sc as plsc

_D = 32
_CHUNK = 256
_NW = 32
_TROWS = 1564
_OFF_BLACK = 768 * _D
_OFF_SMALL = (1536 * _D, 1540 * _D, 1544 * _D, 1552 * _D, 1560 * _D,
              1562 * _D)


def _sc_kernel(wpi_hbm, bpi_hbm, sm_hbm, table_hbm, out_hbm,
               table_v, wpi_v, bpi_v, sm_v, out_v, acc_v):
    B = out_hbm.shape[0] // _D
    per_w = B // _NW
    wid = lax.axis_index("s") * 2 + lax.axis_index("c")
    it16 = jnp.arange(16, dtype=jnp.int32)

    pltpu.sync_copy(table_hbm, table_v)

    def do_group(g, carry):
        lanes = g * 16 + it16

        def make_body(idx_ref, off):
            def body(s, accs):
                idxv = plsc.load_gather(idx_ref, [lanes * 64 + s])
                wordv = idxv * _D + (s * (12 * _D) + off)
                return tuple(
                    acc + plsc.load_gather(table_v, [wordv + d])
                    for d, acc in enumerate(accs))
            return body

        accs = tuple(jnp.zeros((16,), jnp.float32) for _ in range(_D))
        accs = lax.fori_loop(0, 64, make_body(wpi_v, 0), accs)
        accs = lax.fori_loop(0, 64, make_body(bpi_v, _OFF_BLACK), accs)

        for s, off in enumerate(_OFF_SMALL):
            idxv = plsc.load_gather(sm_v, [lanes * 6 + s])
            wordv = idxv * _D + off
            accs = tuple(
                acc + plsc.load_gather(table_v, [wordv + d])
                for d, acc in enumerate(accs))

        for d in range(_D):
            plsc.store_scatter(out_v, [lanes * _D + d], accs[d])
        return carry

    for chunk in range(per_w // _CHUNK):
        base = wid * per_w + chunk * _CHUNK
        pltpu.sync_copy(wpi_hbm.at[pl.ds(base * 64, _CHUNK * 64)], wpi_v)
        pltpu.sync_copy(bpi_hbm.at[pl.ds(base * 64, _CHUNK * 64)], bpi_v)
        pltpu.sync_copy(sm_hbm.at[pl.ds(base * 6, _CHUNK * 6)], sm_v)
        lax.fori_loop(0, _CHUNK // 16, do_group, 0)
        pltpu.sync_copy(out_v, out_hbm.at[pl.ds(base * _D, _CHUNK * _D)])


def kernel(white_piece_idx, black_piece_idx, white_castle_idx,
           black_castle_idx, white_ep_idx, black_ep_idx, white_fifty_idx,
           black_fifty_idx, W_white_piece, W_black_piece, W_white_castle,
           W_black_castle, W_white_ep, W_black_ep, W_white_fifty,
           W_black_fifty):
    B = white_piece_idx.shape[0]

    table = jnp.concatenate(
        [W_white_piece.reshape(768, _D), W_black_piece.reshape(768, _D),
         W_white_castle, W_black_castle, W_white_ep, W_black_ep,
         W_white_fifty, W_black_fifty], axis=0).reshape(-1)

    sm = jnp.stack(
        [white_castle_idx, black_castle_idx, white_ep_idx, black_ep_idx,
         white_fifty_idx, black_fifty_idx], axis=1).astype(jnp.int32)

    mesh = plsc.VectorSubcoreMesh(core_axis_name="c", subcore_axis_name="s")
    run = functools.partial(
        pl.kernel, mesh=mesh,
        compiler_params=pltpu.CompilerParams(needs_layout_passes=False),
        out_type=jax.ShapeDtypeStruct((B * _D,), jnp.float32),
        scratch_types=[
            pltpu.VMEM((_TROWS * _D,), jnp.float32),
            pltpu.VMEM((_CHUNK * 64,), jnp.int32),
            pltpu.VMEM((_CHUNK * 64,), jnp.int32),
            pltpu.VMEM((_CHUNK * 6,), jnp.int32),
            pltpu.VMEM((_CHUNK * _D,), jnp.float32),
            pltpu.VMEM((_D * 16,), jnp.float32),
        ],
    )(_sc_kernel)
    out = run(white_piece_idx.astype(jnp.int32).reshape(-1),
              black_piece_idx.astype(jnp.int32).reshape(-1),
              sm.reshape(-1), table)
    return out.reshape(B, _D)


# speedup vs baseline: 1.7067x; 1.0144x over previous
import functools

import jax
import jax.numpy as jnp
from jax import lax
from jax.experimental import pallas as pl
from jax.experimental.pallas import tpu as pltpu
from jax.experimental.pallas import tpu_sc as plsc

_D = 32
_CHUNK = 256
_NW = 32
_TROWS = 1564
_OFF_BLACK = 768 * _D
_OFF_SMALL = (1536 * _D, 1540 * _D, 1544 * _D, 1552 * _D, 1560 * _D,
              1562 * _D)


def _sc_kernel(wpi_hbm, bpi_hbm, sm_hbm, table_hbm, out_hbm,
               table_v, wpi_v, bpi_v, sm_v, out_v, acc_v):
    B = out_hbm.shape[0] // _D
    per_w = B // _NW
    wid = lax.axis_index("s") * 2 + lax.axis_index("c")
    it16 = jnp.arange(16, dtype=jnp.int32)

    pltpu.sync_copy(table_hbm, table_v)

    _DB = 8

    def do_group(g, carry):
        lanes = g * 16 + it16

        def make_body(idx_ref, off, d0):
            def body(s, accs):
                idxv = plsc.load_gather(idx_ref, [lanes * 64 + s])
                wordv = idxv * _D + (s * (12 * _D) + off + d0)
                return tuple(
                    acc + plsc.load_gather(table_v, [wordv + d])
                    for d, acc in enumerate(accs))
            return body

        for d0 in range(0, _D, _DB):
            accs = tuple(jnp.zeros((16,), jnp.float32) for _ in range(_DB))
            accs = lax.fori_loop(0, 64, make_body(wpi_v, 0, d0), accs)
            accs = lax.fori_loop(0, 64, make_body(bpi_v, _OFF_BLACK, d0),
                                 accs)
            for s, off in enumerate(_OFF_SMALL):
                idxv = plsc.load_gather(sm_v, [lanes * 6 + s])
                wordv = idxv * _D + (off + d0)
                accs = tuple(
                    acc + plsc.load_gather(table_v, [wordv + d])
                    for d, acc in enumerate(accs))
            for d, acc in enumerate(accs):
                plsc.store_scatter(out_v, [lanes * _D + (d0 + d)], acc)
        return carry

    for chunk in range(per_w // _CHUNK):
        base = wid * per_w + chunk * _CHUNK
        pltpu.sync_copy(wpi_hbm.at[pl.ds(base * 64, _CHUNK * 64)], wpi_v)
        pltpu.sync_copy(bpi_hbm.at[pl.ds(base * 64, _CHUNK * 64)], bpi_v)
        pltpu.sync_copy(sm_hbm.at[pl.ds(base * 6, _CHUNK * 6)], sm_v)
        lax.fori_loop(0, _CHUNK // 16, do_group, 0)
        pltpu.sync_copy(out_v, out_hbm.at[pl.ds(base * _D, _CHUNK * _D)])


def kernel(white_piece_idx, black_piece_idx, white_castle_idx,
           black_castle_idx, white_ep_idx, black_ep_idx, white_fifty_idx,
           black_fifty_idx, W_white_piece, W_black_piece, W_white_castle,
           W_black_castle, W_white_ep, W_black_ep, W_white_fifty,
           W_black_fifty):
    B = white_piece_idx.shape[0]

    table = jnp.concatenate(
        [W_white_piece.reshape(768, _D), W_black_piece.reshape(768, _D),
         W_white_castle, W_black_castle, W_white_ep, W_black_ep,
         W_white_fifty, W_black_fifty], axis=0).reshape(-1)

    sm = jnp.stack(
        [white_castle_idx, black_castle_idx, white_ep_idx, black_ep_idx,
         white_fifty_idx, black_fifty_idx], axis=1).astype(jnp.int32)

    mesh = plsc.VectorSubcoreMesh(core_axis_name="c", subcore_axis_name="s")
    run = functools.partial(
        pl.kernel, mesh=mesh,
        compiler_params=pltpu.CompilerParams(needs_layout_passes=False),
        out_type=jax.ShapeDtypeStruct((B * _D,), jnp.float32),
        scratch_types=[
            pltpu.VMEM((_TROWS * _D,), jnp.float32),
            pltpu.VMEM((_CHUNK * 64,), jnp.int32),
            pltpu.VMEM((_CHUNK * 64,), jnp.int32),
            pltpu.VMEM((_CHUNK * 6,), jnp.int32),
            pltpu.VMEM((_CHUNK * _D,), jnp.float32),
            pltpu.VMEM((_D * 16,), jnp.float32),
        ],
    )(_sc_kernel)
    out = run(white_piece_idx.astype(jnp.int32).reshape(-1),
              black_piece_idx.astype(jnp.int32).reshape(-1),
              sm.reshape(-1), table)
    return out.reshape(B, _D)


# speedup vs baseline: 5.7492x; 3.3686x over previous
import functools

import jax
import jax.numpy as jnp
from jax import lax
from jax.experimental import pallas as pl
from jax.experimental.pallas import tpu as pltpu
from jax.experimental.pallas import tpu_sc as plsc

_D = 32
_CHUNK = 256
_NW = 32
_TROWS = 1564
_RS = 33
_OFF_BLACK = 768 * _RS
_OFF_SMALL = (1536 * _RS, 1540 * _RS, 1544 * _RS, 1552 * _RS, 1560 * _RS,
              1562 * _RS)


def _sc_kernel(wpi_hbm, bpi_hbm, sm_hbm, table_hbm, out_hbm,
               table_v, wpi_v, bpi_v, sm_v, out_v, acc_v):
    B = out_hbm.shape[0] // _D
    per_w = B // _NW
    wid = lax.axis_index("s") * 2 + lax.axis_index("c")
    it16 = jnp.arange(16, dtype=jnp.int32)

    pltpu.sync_copy(table_hbm, table_v)

    _DB = 8

    def do_group(g, carry):
        lanes = g * 16 + it16

        def make_body(idx_ref, off, d0):
            def body(s, accs):
                idxv = plsc.load_gather(idx_ref, [lanes * 64 + s])
                wordv = idxv * _RS + (s * (12 * _RS) + off + d0)
                return tuple(
                    acc + plsc.load_gather(table_v, [wordv + d])
                    for d, acc in enumerate(accs))
            return body

        for d0 in range(0, _D, _DB):
            accs = tuple(jnp.zeros((16,), jnp.float32) for _ in range(_DB))
            accs = lax.fori_loop(0, 64, make_body(wpi_v, 0, d0), accs)
            accs = lax.fori_loop(0, 64, make_body(bpi_v, _OFF_BLACK, d0),
                                 accs)
            for s, off in enumerate(_OFF_SMALL):
                idxv = plsc.load_gather(sm_v, [lanes * 6 + s])
                wordv = idxv * _RS + (off + d0)
                accs = tuple(
                    acc + plsc.load_gather(table_v, [wordv + d])
                    for d, acc in enumerate(accs))
            for d, acc in enumerate(accs):
                plsc.store_scatter(out_v, [lanes * _D + (d0 + d)], acc)
        return carry

    for chunk in range(per_w // _CHUNK):
        base = wid * per_w + chunk * _CHUNK
        pltpu.sync_copy(wpi_hbm.at[pl.ds(base * 64, _CHUNK * 64)], wpi_v)
        pltpu.sync_copy(bpi_hbm.at[pl.ds(base * 64, _CHUNK * 64)], bpi_v)
        pltpu.sync_copy(sm_hbm.at[pl.ds(base * 6, _CHUNK * 6)], sm_v)
        lax.fori_loop(0, _CHUNK // 16, do_group, 0)
        pltpu.sync_copy(out_v, out_hbm.at[pl.ds(base * _D, _CHUNK * _D)])


def kernel(white_piece_idx, black_piece_idx, white_castle_idx,
           black_castle_idx, white_ep_idx, black_ep_idx, white_fifty_idx,
           black_fifty_idx, W_white_piece, W_black_piece, W_white_castle,
           W_black_castle, W_white_ep, W_black_ep, W_white_fifty,
           W_black_fifty):
    B = white_piece_idx.shape[0]

    table = jnp.concatenate(
        [W_white_piece.reshape(768, _D), W_black_piece.reshape(768, _D),
         W_white_castle, W_black_castle, W_white_ep, W_black_ep,
         W_white_fifty, W_black_fifty], axis=0)
    table = jnp.pad(table, ((0, 0), (0, _RS - _D))).reshape(-1)

    sm = jnp.stack(
        [white_castle_idx, black_castle_idx, white_ep_idx, black_ep_idx,
         white_fifty_idx, black_fifty_idx], axis=1).astype(jnp.int32)

    mesh = plsc.VectorSubcoreMesh(core_axis_name="c", subcore_axis_name="s")
    run = functools.partial(
        pl.kernel, mesh=mesh,
        compiler_params=pltpu.CompilerParams(needs_layout_passes=False),
        out_type=jax.ShapeDtypeStruct((B * _D,), jnp.float32),
        scratch_types=[
            pltpu.VMEM((_TROWS * _RS,), jnp.float32),
            pltpu.VMEM((_CHUNK * 64,), jnp.int32),
            pltpu.VMEM((_CHUNK * 64,), jnp.int32),
            pltpu.VMEM((_CHUNK * 6,), jnp.int32),
            pltpu.VMEM((_CHUNK * _D,), jnp.float32),
            pltpu.VMEM((_D * 16,), jnp.float32),
        ],
    )(_sc_kernel)
    out = run(white_piece_idx.astype(jnp.int32).reshape(-1),
              black_piece_idx.astype(jnp.int32).reshape(-1),
              sm.reshape(-1), table)
    return out.reshape(B, _D)


# speedup vs baseline: 12.7477x; 2.2173x over previous
import functools

import jax
import jax.numpy as jnp
from jax import lax
from jax.experimental import pallas as pl
from jax.experimental.pallas import tpu as pltpu
from jax.experimental.pallas import tpu_sc as plsc

_D = 32
_W = 16
_RS = 17
_CHUNK = 256
_NW = 32
_TROWS = 1564
_OFF_BLACK = 768 * _RS
_OFF_SMALL = (1536 * _RS, 1540 * _RS, 1544 * _RS, 1552 * _RS, 1560 * _RS,
              1562 * _RS)
_DBW = 4


def _sc_kernel(wpi_hbm, bpi_hbm, sm_hbm, table_hbm, out_hbm,
               table_v, wpi_v, bpi_v, sm_v, out_v):
    B = out_hbm.shape[0] // _D
    per_w = B // _NW
    wid = lax.axis_index("s") * 2 + lax.axis_index("c")
    it16 = jnp.arange(16, dtype=jnp.int32)

    pltpu.sync_copy(table_hbm, table_v)

    def do_group(g, carry):
        lanes = g * 16 + it16

        def gather_row(wordv, w0, accs):
            out = []
            for i in range(_DBW):
                word = plsc.load_gather(table_v, [wordv + (w0 + i)])
                lo, hi = plsc.unpack(plsc.bitcast(word, jnp.bfloat16),
                                     format=plsc.PackFormat.INTERLEAVED)
                out.append(accs[2 * i] + lo)
                out.append(accs[2 * i + 1] + hi)
            return tuple(out)

        def make_body(idx_ref, off, w0):
            def body(s, accs):
                idxv = idx_ref[s, pl.ds(g * 16, 16)]
                wordv = idxv * _RS + (s * (12 * _RS) + off)
                return gather_row(wordv, w0, accs)
            return body

        for w0 in range(0, _W, _DBW):
            accs = tuple(jnp.zeros((16,), jnp.float32)
                         for _ in range(2 * _DBW))
            accs = lax.fori_loop(0, 64, make_body(wpi_v, 0, w0), accs)
            accs = lax.fori_loop(0, 64, make_body(bpi_v, _OFF_BLACK, w0),
                                 accs)
            for s, off in enumerate(_OFF_SMALL):
                idxv = sm_v[s, pl.ds(g * 16, 16)]
                accs = gather_row(idxv * _RS + off, w0, accs)
            for i, acc in enumerate(accs):
                d = 2 * w0 + i
                plsc.store_scatter(out_v, [lanes * _D + d], acc)
        return carry

    for chunk in range(per_w // _CHUNK):
        base = wid * per_w + chunk * _CHUNK
        pltpu.sync_copy(wpi_hbm.at[:, pl.ds(base, _CHUNK)], wpi_v)
        pltpu.sync_copy(bpi_hbm.at[:, pl.ds(base, _CHUNK)], bpi_v)
        pltpu.sync_copy(sm_hbm.at[:, pl.ds(base, _CHUNK)], sm_v)
        lax.fori_loop(0, _CHUNK // 16, do_group, 0)
        pltpu.sync_copy(out_v, out_hbm.at[pl.ds(base * _D, _CHUNK * _D)])


def kernel(white_piece_idx, black_piece_idx, white_castle_idx,
           black_castle_idx, white_ep_idx, black_ep_idx, white_fifty_idx,
           black_fifty_idx, W_white_piece, W_black_piece, W_white_castle,
           W_black_castle, W_white_ep, W_black_ep, W_white_fifty,
           W_black_fifty):
    B = white_piece_idx.shape[0]

    table = jnp.concatenate(
        [W_white_piece.reshape(768, _D), W_black_piece.reshape(768, _D),
         W_white_castle, W_black_castle, W_white_ep, W_black_ep,
         W_white_fifty, W_black_fifty], axis=0)
    tw = jax.lax.bitcast_convert_type(
        table.astype(jnp.bfloat16).reshape(_TROWS, _W, 2), jnp.int32)
    tw = jnp.pad(tw, ((0, 0), (0, _RS - _W))).reshape(-1)

    sm = jnp.stack(
        [white_castle_idx, black_castle_idx, white_ep_idx, black_ep_idx,
         white_fifty_idx, black_fifty_idx], axis=0).astype(jnp.int32)

    mesh = plsc.VectorSubcoreMesh(core_axis_name="c", subcore_axis_name="s")
    run = functools.partial(
        pl.kernel, mesh=mesh,
        compiler_params=pltpu.CompilerParams(needs_layout_passes=False),
        out_type=jax.ShapeDtypeStruct((B * _D,), jnp.float32),
        scratch_types=[
            pltpu.VMEM((_TROWS * _RS,), jnp.int32),
            pltpu.VMEM((64, _CHUNK), jnp.int32),
            pltpu.VMEM((64, _CHUNK), jnp.int32),
            pltpu.VMEM((6, _CHUNK), jnp.int32),
            pltpu.VMEM((_CHUNK * _D,), jnp.float32),
        ],
    )(_sc_kernel)
    out = run(white_piece_idx.astype(jnp.int32).T,
              black_piece_idx.astype(jnp.int32).T, sm, tw)
    return out.reshape(B, _D)


# speedup vs baseline: 13.7650x; 1.0798x over previous
import functools

import jax
import jax.numpy as jnp
from jax import lax
from jax.experimental import pallas as pl
from jax.experimental.pallas import tpu as pltpu
from jax.experimental.pallas import tpu_sc as plsc

_D = 32
_W = 16
_RS = 17
_CHUNK = 256
_NW = 32
_TROWS = 1564
_OFF_BLACK = 768 * _RS
_OFF_SMALL = (1536 * _RS, 1540 * _RS, 1544 * _RS, 1552 * _RS, 1560 * _RS,
              1562 * _RS)
_DBW = 8


def _sc_kernel(wpi_hbm, bpi_hbm, sm_hbm, table_hbm, out_hbm,
               table_v, wpi_v, bpi_v, sm_v, out_v):
    B = out_hbm.shape[0] // _D
    per_w = B // _NW
    wid = lax.axis_index("s") * 2 + lax.axis_index("c")
    it16 = jnp.arange(16, dtype=jnp.int32)

    pltpu.sync_copy(table_hbm, table_v)

    def do_group(g, carry):
        lanes = g * 16 + it16

        def gather_row(wordv, w0, accs):
            out = []
            for i in range(_DBW):
                word = plsc.load_gather(table_v, [wordv + (w0 + i)])
                lo, hi = plsc.unpack(plsc.bitcast(word, jnp.bfloat16),
                                     format=plsc.PackFormat.INTERLEAVED)
                out.append(accs[2 * i] + lo)
                out.append(accs[2 * i + 1] + hi)
            return tuple(out)

        def make_body(idx_ref, off, w0):
            def body(s, accs):
                idxv = idx_ref[s, pl.ds(g * 16, 16)]
                wordv = idxv * _RS + (s * (12 * _RS) + off)
                return gather_row(wordv, w0, accs)
            return body

        for w0 in range(0, _W, _DBW):
            accs = tuple(jnp.zeros((16,), jnp.float32)
                         for _ in range(2 * _DBW))
            accs = lax.fori_loop(0, 64, make_body(wpi_v, 0, w0), accs,
                                 unroll=2)
            accs = lax.fori_loop(0, 64, make_body(bpi_v, _OFF_BLACK, w0),
                                 accs, unroll=2)
            for s, off in enumerate(_OFF_SMALL):
                idxv = sm_v[s, pl.ds(g * 16, 16)]
                accs = gather_row(idxv * _RS + off, w0, accs)
            for i, acc in enumerate(accs):
                d = 2 * w0 + i
                plsc.store_scatter(out_v, [lanes * _D + d], acc)
        return carry

    for chunk in range(per_w // _CHUNK):
        base = wid * per_w + chunk * _CHUNK
        pltpu.sync_copy(wpi_hbm.at[:, pl.ds(base, _CHUNK)], wpi_v)
        pltpu.sync_copy(bpi_hbm.at[:, pl.ds(base, _CHUNK)], bpi_v)
        pltpu.sync_copy(sm_hbm.at[:, pl.ds(base, _CHUNK)], sm_v)
        lax.fori_loop(0, _CHUNK // 16, do_group, 0)
        pltpu.sync_copy(out_v, out_hbm.at[pl.ds(base * _D, _CHUNK * _D)])


def kernel(white_piece_idx, black_piece_idx, white_castle_idx,
           black_castle_idx, white_ep_idx, black_ep_idx, white_fifty_idx,
           black_fifty_idx, W_white_piece, W_black_piece, W_white_castle,
           W_black_castle, W_white_ep, W_black_ep, W_white_fifty,
           W_black_fifty):
    B = white_piece_idx.shape[0]

    table = jnp.concatenate(
        [W_white_piece.reshape(768, _D), W_black_piece.reshape(768, _D),
         W_white_castle, W_black_castle, W_white_ep, W_black_ep,
         W_white_fifty, W_black_fifty], axis=0)
    tw = jax.lax.bitcast_convert_type(
        table.astype(jnp.bfloat16).reshape(_TROWS, _W, 2), jnp.int32)
    tw = jnp.pad(tw, ((0, 0), (0, _RS - _W))).reshape(-1)

    sm = jnp.stack(
        [white_castle_idx, black_castle_idx, white_ep_idx, black_ep_idx,
         white_fifty_idx, black_fifty_idx], axis=0).astype(jnp.int32)

    mesh = plsc.VectorSubcoreMesh(core_axis_name="c", subcore_axis_name="s")
    run = functools.partial(
        pl.kernel, mesh=mesh,
        compiler_params=pltpu.CompilerParams(needs_layout_passes=False),
        out_type=jax.ShapeDtypeStruct((B * _D,), jnp.float32),
        scratch_types=[
            pltpu.VMEM((_TROWS * _RS,), jnp.int32),
            pltpu.VMEM((64, _CHUNK), jnp.int32),
            pltpu.VMEM((64, _CHUNK), jnp.int32),
            pltpu.VMEM((6, _CHUNK), jnp.int32),
            pltpu.VMEM((_CHUNK * _D,), jnp.float32),
        ],
    )(_sc_kernel)
    out = run(white_piece_idx.astype(jnp.int32).T,
              black_piece_idx.astype(jnp.int32).T, sm, tw)
    return out.reshape(B, _D)
